# Initial kernel scaffold; baseline (speedup 1.0000x reference)
import jax
import jax.numpy as jnp
from jax.experimental import pallas as pl


def kernel(x, d, voxel_grid):
    raise NotImplementedError("write your pallas kernel here")


# speedup vs baseline: 4.3283x; 4.3283x over previous
import functools

import jax
import jax.numpy as jnp
from jax import lax
from jax.experimental import pallas as pl
from jax.experimental.pallas import tpu as pltpu
from jax.experimental.pallas import tpu_sc as plsc

_C0 = 0.28209479177387814
_C1 = 0.4886025119029199
_C2 = (1.0925484305920792, -1.0925484305920792, 0.31539156525252005,
       -1.0925484305920792, 0.5462742152960396)

_B = 1048576
_N = 256
_NC = 2
_NS = 16
_NW = _NC * _NS
_BPW = _B // _NW
_CH = 1024
_PB = _CH // 128
_PLANE = _N * _N
_XSTRIDE = 28 * _PLANE


def _sc_body(x0_h, x1_h, x2_h, d0_h, d1_h, d2_h, tab_hbm,
             c0_hbm, c1_hbm, c2_hbm, sig_hbm,
             x0_v, x1_v, x2_v, d0_v, d1_v, d2_v,
             idx_v, cad_v, cpid_v, rows_v, c0_v, c1_v, c2_v, sig_v, sem):
    wid = lax.axis_index("s") * _NC + lax.axis_index("c")
    lane = jnp.arange(16, dtype=jnp.int32)
    zero16 = jnp.zeros((16,), jnp.float32)

    def chunk(t, carry):
        base = (wid * (_BPW // _CH) + t) * _CH
        for src, dst in ((x0_h, x0_v), (x1_h, x1_v), (x2_h, x2_v),
                         (d0_h, d0_v), (d1_h, d1_v), (d2_h, d2_v)):
            pltpu.sync_copy(src.at[pl.ds(base, _CH)], dst)

        def prefill(g, c):
            cad_v[pl.ds(g * 16, 16)] = wid * 1024 + g * 16 + lane
            return c

        lax.fori_loop(0, _CH // 16, prefill, 0)

        def idx_group(g, off):
            sl = pl.ds(g * 16, 16)
            x0 = x0_v[sl]
            x1 = x1_v[sl]
            x2 = x2_v[sl]
            i0 = jnp.clip((x0 * 128.0 + 128.0).astype(jnp.int32), 0, _N - 1)
            i1 = jnp.clip((x1 * 128.0 + 128.0).astype(jnp.int32), 0, _N - 1)
            i2 = jnp.clip((x2 * 128.0 + 128.0).astype(jnp.int32), 0, _N - 1)
            addr = (i0 * _XSTRIDE
                    + ((i1 >> 3) << 11) + ((i2 >> 7) << 10)
                    + ((i1 & 7) << 7) + (i2 & 127))
            inside = ((jnp.abs(x0) < 1.0) & (jnp.abs(x1) < 1.0)
                      & (jnp.abs(x2) < 1.0))
            c0_v[sl] = zero16
            c1_v[sl] = zero16
            c2_v[sl] = zero16
            sig_v[sl] = zero16
            cum = plsc.cumsum(jnp.where(inside, 1, 0).astype(jnp.int32))
            pos = off + cum - 1
            plsc.store_scatter(cad_v, [pos], addr, mask=inside)
            plsc.store_scatter(cpid_v, [pos], g * 16 + lane, mask=inside)
            return off + jnp.max(cum)

        n_in = lax.fori_loop(0, _CH // 16, idx_group, jnp.int32(0))
        nblk = (n_in + 127) >> 7

        def build_block(b, c):
            def build_sub(s, c2):
                v = cad_v[pl.ds(b * 128 + s * 16, 16)]
                for ch in range(28):
                    idx_v.at[b * 28 + ch][pl.ds(s * 16, 16)] = v + ch * _PLANE
                return c2

            lax.fori_loop(0, 8, build_sub, 0)
            return c

        lax.fori_loop(0, nblk, build_block, 0)

        def fire(j, c):
            pltpu.async_copy(tab_hbm.at[idx_v.at[j]],
                             rows_v.at[pl.ds(j * 128, 128)], sem)
            return c

        lax.fori_loop(0, nblk * 28, fire, 0)

        def drain(j, c):
            pltpu.make_async_copy(tab_hbm.at[idx_v.at[0]],
                                  rows_v.at[pl.ds(0, 128)], sem).wait()
            return c

        lax.fori_loop(0, nblk * 28, drain, 0)

        def sh_group(g, c):
            valid = (g * 16 + lane) < n_in
            cpids = cpid_v[pl.ds(g * 16, 16)]
            d0 = plsc.load_gather(d0_v, [cpids], mask=valid)
            d1 = plsc.load_gather(d1_v, [cpids], mask=valid)
            d2 = plsc.load_gather(d2_v, [cpids], mask=valid)
            ss = jnp.maximum(d0 * d0 + d1 * d1 + d2 * d2, 1e-30)
            yi = jnp.int32(0x5F3759DF) - (plsc.bitcast(ss, jnp.int32) >> 1)
            y = plsc.bitcast(yi, jnp.float32)
            y = y * (1.5 - 0.5 * ss * y * y)
            y = y * (1.5 - 0.5 * ss * y * y)
            y = y * (1.5 - 0.5 * ss * y * y)
            inv = 1.0 / (ss * y + 1e-8)
            dx = d0 * inv
            dy = d1 * inv
            dz = d2 * inv
            basis = (_C0,
                     -_C1 * dy,
                     _C1 * dz,
                     -_C1 * dx,
                     _C2[0] * dx * dy,
                     _C2[1] * dy * dz,
                     _C2[2] * (2.0 * dz * dz - dx * dx - dy * dy),
                     _C2[3] * dx * dz,
                     _C2[4] * (dx * dx - dy * dy))
            blk = g >> 3
            w0 = (g & 7) * 16

            def tload(ch):
                return rows_v[pl.ds(blk * 3584 + ch * 128 + w0, 16)]

            plsc.store_scatter(sig_v, [cpids],
                               jnp.maximum(tload(0), 0.0), mask=valid)
            for ch, out_v in ((0, c0_v), (1, c1_v), (2, c2_v)):
                acc = tload(1 + 9 * ch) * basis[0]
                for j in range(1, 9):
                    acc = acc + tload(1 + 9 * ch + j) * basis[j]
                plsc.store_scatter(out_v, [cpids], acc, mask=valid)
            return c

        lax.fori_loop(0, (n_in + 15) >> 4, sh_group, 0)

        for src, dst in ((c0_v, c0_hbm), (c1_v, c1_hbm), (c2_v, c2_hbm),
                         (sig_v, sig_hbm)):
            pltpu.sync_copy(src, dst.at[pl.ds(base, _CH)])
        return carry

    lax.fori_loop(0, _BPW // _CH, chunk, 0)


@functools.partial(
    pl.kernel,
    mesh=plsc.VectorSubcoreMesh(core_axis_name="c", subcore_axis_name="s"),
    out_type=[jax.ShapeDtypeStruct((_B,), jnp.float32),
              jax.ShapeDtypeStruct((_B,), jnp.float32),
              jax.ShapeDtypeStruct((_B,), jnp.float32),
              jax.ShapeDtypeStruct((_B,), jnp.float32)],
    scratch_types=[
        pltpu.VMEM((_CH,), jnp.float32),
        pltpu.VMEM((_CH,), jnp.float32),
        pltpu.VMEM((_CH,), jnp.float32),
        pltpu.VMEM((_CH,), jnp.float32),
        pltpu.VMEM((_CH,), jnp.float32),
        pltpu.VMEM((_CH,), jnp.float32),
        pltpu.VMEM((_PB * 28, 128), jnp.int32),
        pltpu.VMEM((_CH,), jnp.int32),
        pltpu.VMEM((_CH,), jnp.int32),
        pltpu.VMEM((_PB * 28 * 128,), jnp.float32),
        pltpu.VMEM((_CH,), jnp.float32),
        pltpu.VMEM((_CH,), jnp.float32),
        pltpu.VMEM((_CH,), jnp.float32),
        pltpu.VMEM((_CH,), jnp.float32),
        pltpu.SemaphoreType.DMA,
    ],
    compiler_params=pltpu.CompilerParams(needs_layout_passes=False,
                                         use_tc_tiling_on_sc=False),
)
def _plenoxel_sc(*refs):
    _sc_body(*refs)


def kernel(x, d, voxel_grid):
    t = voxel_grid.transpose(0, 3, 1, 2)
    t = t.reshape(_N, 28, _N // 8, 8, 2, 128)
    t = t.transpose(0, 1, 2, 4, 3, 5)
    tab1d = t.reshape(-1)
    c0, c1, c2, sigma = _plenoxel_sc(x[:, 0], x[:, 1], x[:, 2],
                                     d[:, 0], d[:, 1], d[:, 2], tab1d)
    color = jnp.stack([c0, c1, c2], axis=1)
    return (color, sigma)
